# Initial kernel scaffold; baseline (speedup 1.0000x reference)
import jax
import jax.numpy as jnp
from jax.experimental import pallas as pl


def kernel(vertex_features, edges, weights, W1, b1, W2, b2, W3, b3, W4, b4, Wr, br):
    raise NotImplementedError("write your pallas kernel here")


# speedup vs baseline: 1.9437x; 1.9437x over previous
import jax
import jax.numpy as jnp
from jax.experimental import pallas as pl
from jax.experimental.pallas import tpu as pltpu

_N = 100000
_H = 64


def _pool_readout_body(h_ref, wr_ref, br_ref, o_ref, acc_ref):
    i = pl.program_id(0)

    @pl.when(i == 0)
    def _init():
        acc_ref[...] = jnp.full_like(acc_ref, -jnp.inf)

    acc_ref[...] = jnp.maximum(acc_ref[...], jnp.max(h_ref[...], axis=0, keepdims=True))

    @pl.when(i == pl.num_programs(0) - 1)
    def _fin():
        o_ref[...] = jnp.sum(acc_ref[...] * wr_ref[...], axis=1, keepdims=True) + br_ref[...]


def kernel(vertex_features, edges, weights, W1, b1, W2, b2, W3, b3, W4, b4, Wr, br):
    src = edges[0]
    dst = edges[1]

    deg = jnp.zeros((_N,), jnp.float32).at[dst].add(weights) + 1.0
    dinv = jax.lax.rsqrt(deg)

    h = vertex_features
    for (W, b) in ((W1, b1), (W2, b2), (W3, b3), (W4, b4)):
        y = dinv[:, None] * (h @ W)
        acc = jnp.zeros_like(y).at[dst].add(weights[:, None] * y[src])
        h = jax.nn.relu(dinv[:, None] * (acc + y) + b)

    blk = 10000
    out = pl.pallas_call(
        _pool_readout_body,
        grid=(_N // blk,),
        in_specs=[
            pl.BlockSpec((blk, _H), lambda i: (i, 0)),
            pl.BlockSpec((1, _H), lambda i: (0, 0)),
            pl.BlockSpec((1, 1), lambda i: (0, 0)),
        ],
        out_specs=pl.BlockSpec((1, 1), lambda i: (0, 0)),
        out_shape=jax.ShapeDtypeStruct((1, 1), jnp.float32),
        scratch_shapes=[pltpu.VMEM((1, _H), jnp.float32)],
    )(h, Wr.reshape(1, _H), br.reshape(1, 1))
    return jnp.squeeze(out)


# speedup vs baseline: 5.7788x; 2.9730x over previous
import functools

import jax
import jax.numpy as jnp
from jax import lax
from jax.experimental import pallas as pl
from jax.experimental.pallas import tpu as pltpu
from jax.experimental.pallas import tpu_sc as plsc

N = 100000
E = 1600000
H = 64
HC = 16
NSUB = 16
NPAD = 102400
NT = NPAD // NSUB
EB = 800
ET = E // NSUB
ED = E // 2 // NSUB
EBD = 1000

_mesh = plsc.VectorSubcoreMesh(core_axis_name="c", subcore_axis_name="s")


def _zero_vmem2d(ref, nrows):
    z = jnp.zeros((16,), jnp.float32)

    def body(i, carry):
        ref[i, :] = z
        return carry

    lax.fori_loop(0, nrows, body, 0)


def _zero_spmem_slice(acc_sp, zbuf, base, zb):
    nfull = NT // zb
    rem = NT - nfull * zb
    for t in range(nfull):
        pltpu.sync_copy(zbuf, acc_sp.at[pl.ds(base + t * zb, zb), :])
    if rem:
        pltpu.sync_copy(zbuf.at[pl.ds(0, rem), :], acc_sp.at[pl.ds(base + nfull * zb, rem), :])


def _msgpass_body(y4, src_h, dst_h, w_h, acc_out,
                  acc_sp, srcv, dstv, wv, idxv, rowsv, fidxv, sem):
    c = lax.axis_index("c")
    s = lax.axis_index("s")
    iota16 = lax.iota(jnp.int32, 16)
    zero16i = jnp.zeros((16,), jnp.int32)
    ebase = s * ET
    nblocks = ET // EB
    ngroups = EB // 16

    for p in range(2):
        j = c * 2 + p
        _zero_vmem2d(rowsv, EB)
        _zero_spmem_slice(acc_sp, rowsv, s * NT, EB)
        plsc.subcore_barrier()

        def block(b, carry):
            off = ebase + b * EB
            pltpu.sync_copy(src_h.at[pl.ds(off, EB)], srcv)
            pltpu.sync_copy(dst_h.at[pl.ds(off, EB)], dstv)
            pltpu.sync_copy(w_h.at[pl.ds(off, EB)], wv)

            def gidx(g, cg):
                s16 = srcv[pl.ds(g * 16, 16)]
                idxv[pl.ds(g * 16, 16)] = s16 * 4 + j
                return cg

            lax.fori_loop(0, ngroups, gidx, 0)
            pltpu.async_copy(y4.at[idxv], rowsv, sem).wait()

            def scale(i, cg):
                wsp = plsc.load_gather(wv, [zero16i + i])
                rowsv[i, :] = rowsv[i, :] * wsp
                return cg

            lax.fori_loop(0, EB, scale, 0)
            pltpu.sync_copy(rowsv, acc_sp.at[dstv], add=True)
            return carry

        lax.fori_loop(0, nblocks, block, 0)
        plsc.subcore_barrier()
        fbase = s * NT
        for t in range(NT // EB):
            cb = fbase + t * EB

            def fput(g, cg):
                fidxv[pl.ds(g * 16, 16)] = (cb + g * 16 + iota16) * 4 + j
                return cg

            lax.fori_loop(0, EB // 16, fput, 0)
            pltpu.sync_copy(acc_sp.at[pl.ds(cb, EB), :], rowsv)
            pltpu.sync_copy(rowsv, acc_out.at[fidxv])
        plsc.subcore_barrier()


def _msgpass(y4, src, dst, w):
    return pl.kernel(
        _msgpass_body,
        out_type=jax.ShapeDtypeStruct((4 * NPAD, HC), jnp.float32),
        mesh=_mesh,
        compiler_params=pltpu.CompilerParams(needs_layout_passes=False, use_tc_tiling_on_sc=False),
        scratch_types=[
            pltpu.VMEM_SHARED((NPAD, HC), jnp.float32),
            pltpu.VMEM((EB,), jnp.int32),
            pltpu.VMEM((EB,), jnp.int32),
            pltpu.VMEM((EB,), jnp.float32),
            pltpu.VMEM((EB,), jnp.int32),
            pltpu.VMEM((EB, HC), jnp.float32),
            pltpu.VMEM((EB,), jnp.int32),
            pltpu.SemaphoreType.DMA,
        ],
    )(y4, src, dst, w)


def _deg_body(dst_h, w_h, degp_out, deg_sp, dstv, wv, wrows):
    c = lax.axis_index("c")
    s = lax.axis_index("s")
    iota16 = lax.iota(jnp.int32, 16)
    zero16i = jnp.zeros((16,), jnp.int32)
    lane0 = iota16 == 0
    _zero_vmem2d(wrows, EBD)
    _zero_spmem_slice(deg_sp, wrows, s * NT, EBD)
    plsc.subcore_barrier()

    ebase = c * (E // 2) + s * ED
    nblocks = ED // EBD

    def block(b, carry):
        off = ebase + b * EBD
        pltpu.sync_copy(dst_h.at[pl.ds(off, EBD)], dstv)
        pltpu.sync_copy(w_h.at[pl.ds(off, EBD)], wv)

        def put(i, cg):
            wsp = plsc.load_gather(wv, [zero16i + i])
            wrows[i, :] = jnp.where(lane0, wsp, 0.0)
            return cg

        lax.fori_loop(0, EBD, put, 0)
        pltpu.sync_copy(wrows, deg_sp.at[dstv], add=True)
        return carry

    lax.fori_loop(0, nblocks, block, 0)
    plsc.subcore_barrier()
    fbase = s * NT
    pltpu.sync_copy(deg_sp.at[pl.ds(fbase, NT), :],
                    degp_out.at[c, pl.ds(fbase, NT), :])
    plsc.subcore_barrier()


def _deg(dst, w):
    return pl.kernel(
        _deg_body,
        out_type=jax.ShapeDtypeStruct((2, NPAD, HC), jnp.float32),
        mesh=_mesh,
        compiler_params=pltpu.CompilerParams(needs_layout_passes=False, use_tc_tiling_on_sc=False),
        scratch_types=[
            pltpu.VMEM_SHARED((NPAD, HC), jnp.float32),
            pltpu.VMEM((EBD,), jnp.int32),
            pltpu.VMEM((EBD,), jnp.float32),
            pltpu.VMEM((EBD, HC), jnp.float32),
        ],
    )(dst, w)


_BN = 5000


def _dinv_body(degp_ref, dinv_ref):
    d = degp_ref[0, :, 0:1] + degp_ref[1, :, 0:1] + 1.0
    dinv_ref[...] = lax.rsqrt(d)


def _dinv(degp):
    bd = 2000
    return pl.pallas_call(
        _dinv_body,
        grid=(N // bd,),
        in_specs=[pl.BlockSpec((2, bd, HC), lambda i: (0, i, 0))],
        out_specs=pl.BlockSpec((bd, 1), lambda i: (i, 0)),
        out_shape=jax.ShapeDtypeStruct((N, 1), jnp.float32),
    )(degp)


def _layer1_body(x_ref, dinv_ref, w_ref, y_ref):
    y_ref[...] = dinv_ref[...] * jnp.dot(
        x_ref[...], w_ref[...], preferred_element_type=jnp.float32)


def _layer1(x, dinv, W1):
    return pl.pallas_call(
        _layer1_body,
        grid=(N // _BN,),
        in_specs=[
            pl.BlockSpec((_BN, 6), lambda i: (i, 0)),
            pl.BlockSpec((_BN, 1), lambda i: (i, 0)),
            pl.BlockSpec((6, H), lambda i: (0, 0)),
        ],
        out_specs=pl.BlockSpec((_BN, H), lambda i: (i, 0)),
        out_shape=jax.ShapeDtypeStruct((N, H), jnp.float32),
    )(x, dinv, W1)


def _layer_body(acc_ref, y_ref, dinv_ref, b_ref, w_ref, out_ref):
    dinv = dinv_ref[...]
    h = jnp.maximum(dinv * (acc_ref[...] + y_ref[...]) + b_ref[...], 0.0)
    out_ref[...] = dinv * jnp.dot(h, w_ref[...], preferred_element_type=jnp.float32)


def _layer(acc, y, dinv, b, W):
    return pl.pallas_call(
        _layer_body,
        grid=(N // _BN,),
        in_specs=[
            pl.BlockSpec((_BN, H), lambda i: (i, 0)),
            pl.BlockSpec((_BN, H), lambda i: (i, 0)),
            pl.BlockSpec((_BN, 1), lambda i: (i, 0)),
            pl.BlockSpec((1, H), lambda i: (0, 0)),
            pl.BlockSpec((H, H), lambda i: (0, 0)),
        ],
        out_specs=pl.BlockSpec((_BN, H), lambda i: (i, 0)),
        out_shape=jax.ShapeDtypeStruct((N, H), jnp.float32),
    )(acc, y, dinv, b.reshape(1, H), W)


def _readout_body(acc_ref, y_ref, dinv_ref, b_ref, wr_ref, br_ref, o_ref, m_ref):
    i = pl.program_id(0)

    @pl.when(i == 0)
    def _init():
        m_ref[...] = jnp.full_like(m_ref, -jnp.inf)

    h = jnp.maximum(dinv_ref[...] * (acc_ref[...] + y_ref[...]) + b_ref[...], 0.0)
    m_ref[...] = jnp.maximum(m_ref[...], jnp.max(h, axis=0, keepdims=True))

    @pl.when(i == pl.num_programs(0) - 1)
    def _fin():
        o_ref[...] = jnp.sum(m_ref[...] * wr_ref[...], axis=1, keepdims=True) + br_ref[...]


def _readout(acc, y, dinv, b, Wr, br):
    return pl.pallas_call(
        _readout_body,
        grid=(N // _BN,),
        in_specs=[
            pl.BlockSpec((_BN, H), lambda i: (i, 0)),
            pl.BlockSpec((_BN, H), lambda i: (i, 0)),
            pl.BlockSpec((_BN, 1), lambda i: (i, 0)),
            pl.BlockSpec((1, H), lambda i: (0, 0)),
            pl.BlockSpec((1, H), lambda i: (0, 0)),
            pl.BlockSpec((1, 1), lambda i: (0, 0)),
        ],
        out_specs=pl.BlockSpec((1, 1), lambda i: (0, 0)),
        out_shape=jax.ShapeDtypeStruct((1, 1), jnp.float32),
        scratch_shapes=[pltpu.VMEM((1, H), jnp.float32)],
    )(acc, y, dinv, b.reshape(1, H), Wr.reshape(1, H), br.reshape(1, 1))


def kernel(vertex_features, edges, weights, W1, b1, W2, b2, W3, b3, W4, b4, Wr, br):
    src = edges[0]
    dst = edges[1]

    degp = _deg(dst, weights)
    dinv = _dinv(degp)

    y = _layer1(vertex_features, dinv, W1)
    bs = (b1, b2, b3, b4)
    Ws = (None, W2, W3, W4)
    for k in range(4):
        acc = _msgpass(y.reshape(4 * N, HC), src, dst, weights).reshape(NPAD, H)
        if k < 3:
            y = _layer(acc, y, dinv, bs[k], Ws[k + 1])
    out = _readout(acc, y, dinv, b4, Wr, br)
    return jnp.squeeze(out)


# speedup vs baseline: 8.6095x; 1.4898x over previous
import functools

import jax
import jax.numpy as jnp
from jax import lax
from jax.experimental import pallas as pl
from jax.experimental.pallas import tpu as pltpu
from jax.experimental.pallas import tpu_sc as plsc

N = 100000
E = 1600000
H = 64
HC = 16
NSUB = 16
NPAD = 100096
NT = NPAD // NSUB
EB = 800
ET = E // NSUB
ED = E // 2 // NSUB
EBD = 1000

_mesh = plsc.VectorSubcoreMesh(core_axis_name="c", subcore_axis_name="s")


def _zero_vmem2d(ref, nrows):
    z = jnp.zeros((16,), jnp.float32)

    def body(i, carry):
        ref[i, :] = z
        return carry

    lax.fori_loop(0, nrows, body, 0)


def _zero_spmem_slice(acc_sp, zbuf, base, zb):
    nfull = NT // zb
    rem = NT - nfull * zb
    for t in range(nfull):
        pltpu.sync_copy(zbuf, acc_sp.at[pl.ds(base + t * zb, zb), :])
    if rem:
        pltpu.sync_copy(zbuf.at[pl.ds(0, rem), :], acc_sp.at[pl.ds(base + nfull * zb, rem), :])


def _msgpass_body(y4, idx4, dst_h, w_h, acc_out,
                  acc_sp, idx0, idx1, dst0, dst1, w0, w1, rows0, rows1,
                  sg0, sg1, ss0, ss1):
    c = lax.axis_index("c")
    s = lax.axis_index("s")
    iota16 = lax.iota(jnp.int32, 16)
    zero16i = jnp.zeros((16,), jnp.int32)
    ebase = s * ET
    nblocks = ET // EB
    npairs = (nblocks - 1) // 2

    idxs = (idx0, idx1)
    dsts = (dst0, dst1)
    ws = (w0, w1)
    rows = (rows0, rows1)
    sgs = (sg0, sg1)
    sss = (ss0, ss1)

    for p in range(2):
        j = c * 2 + p
        _zero_vmem2d(rows0, EB)
        _zero_spmem_slice(acc_sp, rows0, s * NT, EB)
        plsc.subcore_barrier()

        def stage_fire(off, sl):
            pltpu.async_copy(idx4.at[j, pl.ds(off, EB)], idxs[sl], sss[sl])
            pltpu.async_copy(dst_h.at[pl.ds(off, EB)], dsts[sl], sss[sl])
            pltpu.async_copy(w_h.at[pl.ds(off, EB)], ws[sl], sss[sl])

        def stage_wait(sl):
            pltpu.make_async_copy(idx4.at[j, pl.ds(ebase, EB)], idxs[sl], sss[sl]).wait()
            pltpu.make_async_copy(dst_h.at[pl.ds(ebase, EB)], dsts[sl], sss[sl]).wait()
            pltpu.make_async_copy(w_h.at[pl.ds(ebase, EB)], ws[sl], sss[sl]).wait()

        def gather_fire(sl):
            pltpu.async_copy(y4.at[idxs[sl]], rows[sl], sgs[sl])

        def gather_wait(sl):
            pltpu.make_async_copy(y4.at[idxs[sl]], rows[sl], sgs[sl]).wait()

        def process(sl):
            rv, wvb = rows[sl], ws[sl]

            def scale8(g, cg):
                base = g * 8
                for k in range(8):
                    i = base + k
                    wsp = plsc.load_gather(wvb, [zero16i + i])
                    rv[i, :] = rv[i, :] * wsp
                return cg

            lax.fori_loop(0, EB // 8, scale8, 0)
            pltpu.sync_copy(rv, acc_sp.at[dsts[sl]], add=True)

        stage_fire(ebase, 0)
        stage_wait(0)
        gather_fire(0)
        stage_fire(ebase + EB, 1)

        def pair(pp, carry):
            b0 = pp * 2
            stage_wait(1)
            gather_fire(1)
            gather_wait(0)
            process(0)
            stage_fire(ebase + (b0 + 2) * EB, 0)
            stage_wait(0)
            gather_fire(0)
            gather_wait(1)
            process(1)

            @pl.when(pp < npairs - 1)
            def _():
                stage_fire(ebase + (b0 + 3) * EB, 1)

            return carry

        lax.fori_loop(0, npairs, pair, 0)
        gather_wait(0)
        process(0)

        plsc.subcore_barrier()
        fbase = s * NT
        nf = NT // EB
        rem = NT - nf * EB
        for t in range(nf + (1 if rem else 0)):
            cb = fbase + t * EB
            lim = EB if t < nf else rem

            def fput(g, cg):
                pos16 = g * 16 + iota16
                if lim == EB:
                    node = cb + pos16
                else:
                    node = jnp.where(pos16 < lim, cb + pos16,
                                     N + (pos16 & 63))
                idx0[pl.ds(g * 16, 16)] = node * 4 + j
                return cg

            lax.fori_loop(0, EB // 16, fput, 0)
            pltpu.sync_copy(acc_sp.at[pl.ds(cb, lim), :],
                            rows0.at[pl.ds(0, lim), :])
            pltpu.sync_copy(rows0, acc_out.at[idx0])
        plsc.subcore_barrier()


def _msgpass(y4, idx4, dst, w):
    return pl.kernel(
        _msgpass_body,
        out_type=jax.ShapeDtypeStruct((4 * NPAD, HC), jnp.float32),
        mesh=_mesh,
        compiler_params=pltpu.CompilerParams(needs_layout_passes=False, use_tc_tiling_on_sc=False),
        scratch_types=[
            pltpu.VMEM_SHARED((NPAD, HC), jnp.float32),
            pltpu.VMEM((EB,), jnp.int32),
            pltpu.VMEM((EB,), jnp.int32),
            pltpu.VMEM((EB,), jnp.int32),
            pltpu.VMEM((EB,), jnp.int32),
            pltpu.VMEM((EB,), jnp.float32),
            pltpu.VMEM((EB,), jnp.float32),
            pltpu.VMEM((EB, HC), jnp.float32),
            pltpu.VMEM((EB, HC), jnp.float32),
            pltpu.SemaphoreType.DMA,
            pltpu.SemaphoreType.DMA,
            pltpu.SemaphoreType.DMA,
            pltpu.SemaphoreType.DMA,
        ],
    )(y4, idx4, dst, w)


def _deg_body(dst_h, w_h, degp_out, deg_sp, dstv, wv, wrows):
    c = lax.axis_index("c")
    s = lax.axis_index("s")
    iota16 = lax.iota(jnp.int32, 16)
    zero16i = jnp.zeros((16,), jnp.int32)
    lane0 = iota16 == 0
    _zero_vmem2d(wrows, EBD)
    _zero_spmem_slice(deg_sp, wrows, s * NT, EBD)
    plsc.subcore_barrier()

    ebase = c * (E // 2) + s * ED
    nblocks = ED // EBD

    def block(b, carry):
        off = ebase + b * EBD
        pltpu.sync_copy(dst_h.at[pl.ds(off, EBD)], dstv)
        pltpu.sync_copy(w_h.at[pl.ds(off, EBD)], wv)

        def put(i, cg):
            wsp = plsc.load_gather(wv, [zero16i + i])
            wrows[i, :] = jnp.where(lane0, wsp, 0.0)
            return cg

        lax.fori_loop(0, EBD, put, 0)
        pltpu.sync_copy(wrows, deg_sp.at[dstv], add=True)
        return carry

    lax.fori_loop(0, nblocks, block, 0)
    plsc.subcore_barrier()
    fbase = s * NT
    pltpu.sync_copy(deg_sp.at[pl.ds(fbase, NT), :],
                    degp_out.at[c, pl.ds(fbase, NT), :])
    plsc.subcore_barrier()


def _deg(dst, w):
    return pl.kernel(
        _deg_body,
        out_type=jax.ShapeDtypeStruct((2, NPAD, HC), jnp.float32),
        mesh=_mesh,
        compiler_params=pltpu.CompilerParams(needs_layout_passes=False, use_tc_tiling_on_sc=False),
        scratch_types=[
            pltpu.VMEM_SHARED((NPAD, HC), jnp.float32),
            pltpu.VMEM((EBD,), jnp.int32),
            pltpu.VMEM((EBD,), jnp.float32),
            pltpu.VMEM((EBD, HC), jnp.float32),
        ],
    )(dst, w)


_BN = 5000


def _dinv_body(degp_ref, dinv_ref):
    d = degp_ref[0, :, 0:1] + degp_ref[1, :, 0:1] + 1.0
    dinv_ref[...] = lax.rsqrt(d)


def _dinv(degp):
    bd = 2000
    return pl.pallas_call(
        _dinv_body,
        grid=(N // bd,),
        in_specs=[pl.BlockSpec((2, bd, HC), lambda i: (0, i, 0))],
        out_specs=pl.BlockSpec((bd, 1), lambda i: (i, 0)),
        out_shape=jax.ShapeDtypeStruct((N, 1), jnp.float32),
    )(degp)


def _layer1_body(x_ref, dinv_ref, w_ref, y_ref):
    y_ref[...] = dinv_ref[...] * jnp.dot(
        x_ref[...], w_ref[...], preferred_element_type=jnp.float32)


def _layer1(x, dinv, W1):
    return pl.pallas_call(
        _layer1_body,
        grid=(N // _BN,),
        in_specs=[
            pl.BlockSpec((_BN, 6), lambda i: (i, 0)),
            pl.BlockSpec((_BN, 1), lambda i: (i, 0)),
            pl.BlockSpec((6, H), lambda i: (0, 0)),
        ],
        out_specs=pl.BlockSpec((_BN, H), lambda i: (i, 0)),
        out_shape=jax.ShapeDtypeStruct((N, H), jnp.float32),
    )(x, dinv, W1)


def _layer_body(acc_ref, y_ref, dinv_ref, b_ref, w_ref, out_ref):
    dinv = dinv_ref[...]
    h = jnp.maximum(dinv * (acc_ref[...] + y_ref[...]) + b_ref[...], 0.0)
    out_ref[...] = dinv * jnp.dot(h, w_ref[...], preferred_element_type=jnp.float32)


def _layer(acc, y, dinv, b, W):
    return pl.pallas_call(
        _layer_body,
        grid=(N // _BN,),
        in_specs=[
            pl.BlockSpec((_BN, H), lambda i: (i, 0)),
            pl.BlockSpec((_BN, H), lambda i: (i, 0)),
            pl.BlockSpec((_BN, 1), lambda i: (i, 0)),
            pl.BlockSpec((1, H), lambda i: (0, 0)),
            pl.BlockSpec((H, H), lambda i: (0, 0)),
        ],
        out_specs=pl.BlockSpec((_BN, H), lambda i: (i, 0)),
        out_shape=jax.ShapeDtypeStruct((N, H), jnp.float32),
    )(acc, y, dinv, b.reshape(1, H), W)


def _readout_body(acc_ref, y_ref, dinv_ref, b_ref, wr_ref, br_ref, o_ref, m_ref):
    i = pl.program_id(0)

    @pl.when(i == 0)
    def _init():
        m_ref[...] = jnp.full_like(m_ref, -jnp.inf)

    h = jnp.maximum(dinv_ref[...] * (acc_ref[...] + y_ref[...]) + b_ref[...], 0.0)
    m_ref[...] = jnp.maximum(m_ref[...], jnp.max(h, axis=0, keepdims=True))

    @pl.when(i == pl.num_programs(0) - 1)
    def _fin():
        o_ref[...] = jnp.sum(m_ref[...] * wr_ref[...], axis=1, keepdims=True) + br_ref[...]


def _readout(acc, y, dinv, b, Wr, br):
    return pl.pallas_call(
        _readout_body,
        grid=(N // _BN,),
        in_specs=[
            pl.BlockSpec((_BN, H), lambda i: (i, 0)),
            pl.BlockSpec((_BN, H), lambda i: (i, 0)),
            pl.BlockSpec((_BN, 1), lambda i: (i, 0)),
            pl.BlockSpec((1, H), lambda i: (0, 0)),
            pl.BlockSpec((1, H), lambda i: (0, 0)),
            pl.BlockSpec((1, 1), lambda i: (0, 0)),
        ],
        out_specs=pl.BlockSpec((1, 1), lambda i: (0, 0)),
        out_shape=jax.ShapeDtypeStruct((1, 1), jnp.float32),
        scratch_shapes=[pltpu.VMEM((1, H), jnp.float32)],
    )(acc, y, dinv, b.reshape(1, H), Wr.reshape(1, H), br.reshape(1, 1))


def kernel(vertex_features, edges, weights, W1, b1, W2, b2, W3, b3, W4, b4, Wr, br):
    src = edges[0]
    dst = edges[1]

    degp = _deg(dst, weights)
    dinv = _dinv(degp)

    y = _layer1(vertex_features, dinv, W1)
    bs = (b1, b2, b3, b4)
    Ws = (None, W2, W3, W4)
    idx4 = src[None, :] * 4 + jnp.arange(4, dtype=jnp.int32)[:, None]
    for k in range(4):
        acc = _msgpass(y.reshape(4 * N, HC), idx4, dst, weights).reshape(NPAD, H)
        if k < 3:
            y = _layer(acc, y, dinv, bs[k], Ws[k + 1])
    out = _readout(acc, y, dinv, b4, Wr, br)
    return jnp.squeeze(out)


# speedup vs baseline: 16.1087x; 1.8710x over previous
import functools

import jax
import jax.numpy as jnp
from jax import lax
from jax.experimental import pallas as pl
from jax.experimental.pallas import tpu as pltpu
from jax.experimental.pallas import tpu_sc as plsc

N = 100000
E = 1600000
H = 64
HC = 16
NSUB = 16
NPAD = 100096
NT = NPAD // NSUB
EB = 800
ET = E // NSUB
ED = E // 2 // NSUB
EBD = 1000

_mesh = plsc.VectorSubcoreMesh(core_axis_name="c", subcore_axis_name="s")


def _zero_vmem2d(ref, nrows):
    z = jnp.zeros((16,), jnp.float32)

    def body(i, carry):
        ref[i, :] = z
        return carry

    lax.fori_loop(0, nrows, body, 0)


def _zero_spmem_slice(acc_sp, zbuf, base, zb):
    nfull = NT // zb
    rem = NT - nfull * zb
    for t in range(nfull):
        pltpu.sync_copy(zbuf, acc_sp.at[pl.ds(base + t * zb, zb), :])
    if rem:
        pltpu.sync_copy(zbuf.at[pl.ds(0, rem), :], acc_sp.at[pl.ds(base + nfull * zb, rem), :])


def _msgpass_body(y4, idx4, dst_h, w_h, acc_out,
                  acc_sp, idx0, idx1, dst0, dst1, w0, w1, rows0, rows1,
                  sg0, sg1, ss0, ss1):
    c = lax.axis_index("c")
    s = lax.axis_index("s")
    iota16 = lax.iota(jnp.int32, 16)
    zero16i = jnp.zeros((16,), jnp.int32)
    ebase = s * ET
    nblocks = ET // EB
    npairs = (nblocks - 1) // 2

    idxs = (idx0, idx1)
    dsts = (dst0, dst1)
    ws = (w0, w1)
    rows = (rows0, rows1)
    sgs = (sg0, sg1)
    sss = (ss0, ss1)

    for p in range(2):
        j = c * 2 + p
        _zero_vmem2d(rows0, EB)
        _zero_spmem_slice(acc_sp, rows0, s * NT, EB)
        plsc.subcore_barrier()

        def stage_fire(off, sl):
            pltpu.async_copy(idx4.at[j, pl.ds(off, EB)], idxs[sl], sss[sl])
            pltpu.async_copy(dst_h.at[pl.ds(off, EB)], dsts[sl], sss[sl])
            pltpu.async_copy(w_h.at[pl.ds(off, EB)], ws[sl], sss[sl])

        def stage_wait(sl):
            pltpu.make_async_copy(idx4.at[j, pl.ds(ebase, EB)], idxs[sl], sss[sl]).wait()
            pltpu.make_async_copy(dst_h.at[pl.ds(ebase, EB)], dsts[sl], sss[sl]).wait()
            pltpu.make_async_copy(w_h.at[pl.ds(ebase, EB)], ws[sl], sss[sl]).wait()

        def gather_fire(sl):
            pltpu.async_copy(y4.at[idxs[sl]], rows[sl], sgs[sl])

        def gather_wait(sl):
            pltpu.make_async_copy(y4.at[idxs[sl]], rows[sl], sgs[sl]).wait()

        def process(sl):
            rv, wvb = rows[sl], ws[sl]

            @plsc.parallel_loop(0, EB, step=1, unroll=8)
            def _(i):
                wsp = plsc.load_gather(wvb, [zero16i + i])
                rv[i, :] = rv[i, :] * wsp

            pltpu.sync_copy(rv, acc_sp.at[dsts[sl]], add=True)

        stage_fire(ebase, 0)
        stage_wait(0)
        gather_fire(0)
        stage_fire(ebase + EB, 1)

        def pair(pp, carry):
            b0 = pp * 2
            stage_wait(1)
            gather_fire(1)
            gather_wait(0)
            process(0)
            stage_fire(ebase + (b0 + 2) * EB, 0)
            stage_wait(0)
            gather_fire(0)
            gather_wait(1)
            process(1)

            @pl.when(pp < npairs - 1)
            def _():
                stage_fire(ebase + (b0 + 3) * EB, 1)

            return carry

        lax.fori_loop(0, npairs, pair, 0)
        gather_wait(0)
        process(0)

        plsc.subcore_barrier()
        fbase = s * NT
        nf = NT // EB
        rem = NT - nf * EB
        for t in range(nf + (1 if rem else 0)):
            cb = fbase + t * EB
            lim = EB if t < nf else rem

            def fput(g, cg):
                pos16 = g * 16 + iota16
                if lim == EB:
                    node = cb + pos16
                else:
                    node = jnp.where(pos16 < lim, cb + pos16,
                                     N + (pos16 & 63))
                idx0[pl.ds(g * 16, 16)] = node * 4 + j
                return cg

            lax.fori_loop(0, EB // 16, fput, 0)
            pltpu.sync_copy(acc_sp.at[pl.ds(cb, lim), :],
                            rows0.at[pl.ds(0, lim), :])
            pltpu.sync_copy(rows0, acc_out.at[idx0])
        plsc.subcore_barrier()


def _msgpass(y4, idx4, dst, w):
    return pl.kernel(
        _msgpass_body,
        out_type=jax.ShapeDtypeStruct((4 * NPAD, HC), jnp.float32),
        mesh=_mesh,
        compiler_params=pltpu.CompilerParams(needs_layout_passes=False, use_tc_tiling_on_sc=False),
        scratch_types=[
            pltpu.VMEM_SHARED((NPAD, HC), jnp.float32),
            pltpu.VMEM((EB,), jnp.int32),
            pltpu.VMEM((EB,), jnp.int32),
            pltpu.VMEM((EB,), jnp.int32),
            pltpu.VMEM((EB,), jnp.int32),
            pltpu.VMEM((EB,), jnp.float32),
            pltpu.VMEM((EB,), jnp.float32),
            pltpu.VMEM((EB, HC), jnp.float32),
            pltpu.VMEM((EB, HC), jnp.float32),
            pltpu.SemaphoreType.DMA,
            pltpu.SemaphoreType.DMA,
            pltpu.SemaphoreType.DMA,
            pltpu.SemaphoreType.DMA,
        ],
    )(y4, idx4, dst, w)


def _deg_body(dst_h, w_h, degp_out, deg_sp, dstv, wv, wrows):
    c = lax.axis_index("c")
    s = lax.axis_index("s")
    iota16 = lax.iota(jnp.int32, 16)
    zero16i = jnp.zeros((16,), jnp.int32)
    lane0 = iota16 == 0
    _zero_vmem2d(wrows, EBD)
    _zero_spmem_slice(deg_sp, wrows, s * NT, EBD)
    plsc.subcore_barrier()

    ebase = c * (E // 2) + s * ED
    nblocks = ED // EBD

    def block(b, carry):
        off = ebase + b * EBD
        pltpu.sync_copy(dst_h.at[pl.ds(off, EBD)], dstv)
        pltpu.sync_copy(w_h.at[pl.ds(off, EBD)], wv)

        @plsc.parallel_loop(0, EBD, step=1, unroll=8)
        def _(i):
            wsp = plsc.load_gather(wv, [zero16i + i])
            wrows[i, :] = jnp.where(lane0, wsp, 0.0)
        pltpu.sync_copy(wrows, deg_sp.at[dstv], add=True)
        return carry

    lax.fori_loop(0, nblocks, block, 0)
    plsc.subcore_barrier()
    fbase = s * NT
    pltpu.sync_copy(deg_sp.at[pl.ds(fbase, NT), :],
                    degp_out.at[c, pl.ds(fbase, NT), :])
    plsc.subcore_barrier()


def _deg(dst, w):
    return pl.kernel(
        _deg_body,
        out_type=jax.ShapeDtypeStruct((2, NPAD, HC), jnp.float32),
        mesh=_mesh,
        compiler_params=pltpu.CompilerParams(needs_layout_passes=False, use_tc_tiling_on_sc=False),
        scratch_types=[
            pltpu.VMEM_SHARED((NPAD, HC), jnp.float32),
            pltpu.VMEM((EBD,), jnp.int32),
            pltpu.VMEM((EBD,), jnp.float32),
            pltpu.VMEM((EBD, HC), jnp.float32),
        ],
    )(dst, w)


_BN = 5000


def _dinv_body(degp_ref, dinv_ref):
    d = degp_ref[0, :, 0:1] + degp_ref[1, :, 0:1] + 1.0
    dinv_ref[...] = lax.rsqrt(d)


def _dinv(degp):
    bd = 2000
    return pl.pallas_call(
        _dinv_body,
        grid=(N // bd,),
        in_specs=[pl.BlockSpec((2, bd, HC), lambda i: (0, i, 0))],
        out_specs=pl.BlockSpec((bd, 1), lambda i: (i, 0)),
        out_shape=jax.ShapeDtypeStruct((N, 1), jnp.float32),
    )(degp)


def _layer1_body(x_ref, dinv_ref, w_ref, y_ref):
    y_ref[...] = dinv_ref[...] * jnp.dot(
        x_ref[...], w_ref[...], preferred_element_type=jnp.float32)


def _layer1(x, dinv, W1):
    return pl.pallas_call(
        _layer1_body,
        grid=(N // _BN,),
        in_specs=[
            pl.BlockSpec((_BN, 6), lambda i: (i, 0)),
            pl.BlockSpec((_BN, 1), lambda i: (i, 0)),
            pl.BlockSpec((6, H), lambda i: (0, 0)),
        ],
        out_specs=pl.BlockSpec((_BN, H), lambda i: (i, 0)),
        out_shape=jax.ShapeDtypeStruct((N, H), jnp.float32),
    )(x, dinv, W1)


def _layer_body(acc_ref, y_ref, dinv_ref, b_ref, w_ref, out_ref):
    dinv = dinv_ref[...]
    h = jnp.maximum(dinv * (acc_ref[...] + y_ref[...]) + b_ref[...], 0.0)
    out_ref[...] = dinv * jnp.dot(h, w_ref[...], preferred_element_type=jnp.float32)


def _layer(acc, y, dinv, b, W):
    return pl.pallas_call(
        _layer_body,
        grid=(N // _BN,),
        in_specs=[
            pl.BlockSpec((_BN, H), lambda i: (i, 0)),
            pl.BlockSpec((_BN, H), lambda i: (i, 0)),
            pl.BlockSpec((_BN, 1), lambda i: (i, 0)),
            pl.BlockSpec((1, H), lambda i: (0, 0)),
            pl.BlockSpec((H, H), lambda i: (0, 0)),
        ],
        out_specs=pl.BlockSpec((_BN, H), lambda i: (i, 0)),
        out_shape=jax.ShapeDtypeStruct((N, H), jnp.float32),
    )(acc, y, dinv, b.reshape(1, H), W)


def _readout_body(acc_ref, y_ref, dinv_ref, b_ref, wr_ref, br_ref, o_ref, m_ref):
    i = pl.program_id(0)

    @pl.when(i == 0)
    def _init():
        m_ref[...] = jnp.full_like(m_ref, -jnp.inf)

    h = jnp.maximum(dinv_ref[...] * (acc_ref[...] + y_ref[...]) + b_ref[...], 0.0)
    m_ref[...] = jnp.maximum(m_ref[...], jnp.max(h, axis=0, keepdims=True))

    @pl.when(i == pl.num_programs(0) - 1)
    def _fin():
        o_ref[...] = jnp.sum(m_ref[...] * wr_ref[...], axis=1, keepdims=True) + br_ref[...]


def _readout(acc, y, dinv, b, Wr, br):
    return pl.pallas_call(
        _readout_body,
        grid=(N // _BN,),
        in_specs=[
            pl.BlockSpec((_BN, H), lambda i: (i, 0)),
            pl.BlockSpec((_BN, H), lambda i: (i, 0)),
            pl.BlockSpec((_BN, 1), lambda i: (i, 0)),
            pl.BlockSpec((1, H), lambda i: (0, 0)),
            pl.BlockSpec((1, H), lambda i: (0, 0)),
            pl.BlockSpec((1, 1), lambda i: (0, 0)),
        ],
        out_specs=pl.BlockSpec((1, 1), lambda i: (0, 0)),
        out_shape=jax.ShapeDtypeStruct((1, 1), jnp.float32),
        scratch_shapes=[pltpu.VMEM((1, H), jnp.float32)],
    )(acc, y, dinv, b.reshape(1, H), Wr.reshape(1, H), br.reshape(1, 1))


def kernel(vertex_features, edges, weights, W1, b1, W2, b2, W3, b3, W4, b4, Wr, br):
    src = edges[0]
    dst = edges[1]

    degp = _deg(dst, weights)
    dinv = _dinv(degp)

    y = _layer1(vertex_features, dinv, W1)
    bs = (b1, b2, b3, b4)
    Ws = (None, W2, W3, W4)
    idx4 = src[None, :] * 4 + jnp.arange(4, dtype=jnp.int32)[:, None]
    for k in range(4):
        acc = _msgpass(y.reshape(4 * N, HC), idx4, dst, weights).reshape(NPAD, H)
        if k < 3:
            y = _layer(acc, y, dinv, bs[k], Ws[k + 1])
    out = _readout(acc, y, dinv, b4, Wr, br)
    return jnp.squeeze(out)
